# baseline (device time: 233947 ns/iter reference)
import jax
import jax.numpy as jnp
from jax import lax
from jax.experimental import pallas as pl
from jax.experimental.pallas import tpu as pltpu

N_DEV = 32
MASKS = [16, 4, 2, 1, 8]


def _xorval(s: int) -> int:
    v = 0
    for r in range(5):
        if (s >> r) & 1:
            v ^= MASKS[r]
    return v


def kernel(x, w_mat, scale_x, scale_w):
    m_per, k = x.shape
    n_per = w_mat.shape[1]

    x16 = x.astype(jnp.bfloat16)
    w8 = w_mat.astype(jnp.float8_e4m3fn)

    def body(x_ref, w_ref, sx_ref, sw_ref, out_ref,
             gath, res_buf, rd_send, rd_recv, res_send, res_recv):
        my = lax.axis_index("i")

        barrier_sem = pltpu.get_barrier_semaphore()
        for m in MASKS:
            pl.semaphore_signal(barrier_sem, inc=1, device_id=(my ^ m,),
                                device_id_type=pl.DeviceIdType.MESH)
        pl.semaphore_wait(barrier_sem, len(MASKS))

        scale = sx_ref[0] * sw_ref[0]
        xv = x_ref[...]
        my_rows = pl.ds(my * m_per, m_per)

        def mm(w8_chunk):
            acc = jnp.dot(xv, w8_chunk.astype(jnp.bfloat16),
                          preferred_element_type=jnp.float32)
            return jnp.maximum(acc * scale, 0.0)

        gath[0] = w_ref[...]
        out_ref[my_rows, :] = mm(w_ref[...])

        desc_res = [None] * N_DEV

        def do_block(s):
            lbl = _xorval(s)
            o = my ^ lbl
            res_buf[s] = mm(gath[s])
            desc_res[lbl] = pltpu.make_async_remote_copy(
                src_ref=res_buf.at[s],
                dst_ref=out_ref.at[my_rows, :],
                send_sem=res_send.at[lbl],
                recv_sem=res_recv.at[lbl],
                device_id=(o,),
                device_id_type=pl.DeviceIdType.MESH,
            )
            desc_res[lbl].start()

        desc_rd = []
        for r in range(5):
            sz = 1 << r
            d = pltpu.make_async_remote_copy(
                src_ref=gath.at[pl.ds(0, sz)],
                dst_ref=gath.at[pl.ds(sz, sz)],
                send_sem=rd_send.at[r],
                recv_sem=rd_recv.at[r],
                device_id=(my ^ MASKS[r],),
                device_id_type=pl.DeviceIdType.MESH,
            )
            d.start()
            desc_rd.append(d)
            for s in range(sz >> 1 if r else 1, sz):
                do_block(s)
            d.wait_recv()
        for s in range(16, 32):
            do_block(s)

        for lbl in range(1, N_DEV):
            desc_res[lbl].wait_recv()
        for d in desc_rd:
            d.wait_send()
        for lbl in range(1, N_DEV):
            desc_res[lbl].wait_send()

    return pl.pallas_call(
        body,
        out_shape=jax.ShapeDtypeStruct((N_DEV * m_per, n_per), jnp.float32),
        in_specs=[
            pl.BlockSpec(memory_space=pltpu.VMEM),
            pl.BlockSpec(memory_space=pltpu.VMEM),
            pl.BlockSpec(memory_space=pltpu.SMEM),
            pl.BlockSpec(memory_space=pltpu.SMEM),
        ],
        out_specs=pl.BlockSpec(memory_space=pltpu.VMEM),
        scratch_shapes=[
            pltpu.VMEM((N_DEV, k, n_per), jnp.float8_e4m3fn),
            pltpu.VMEM((N_DEV, m_per, n_per), jnp.float32),
            pltpu.SemaphoreType.DMA((5,)),
            pltpu.SemaphoreType.DMA((5,)),
            pltpu.SemaphoreType.DMA((N_DEV,)),
            pltpu.SemaphoreType.DMA((N_DEV,)),
        ],
        compiler_params=pltpu.CompilerParams(collective_id=0),
    )(x16, w8, scale_x, scale_w)


# device time: 172398 ns/iter; 1.3570x vs baseline; 1.3570x over previous
import jax
import jax.numpy as jnp
from jax import lax
from jax.experimental import pallas as pl
from jax.experimental.pallas import tpu as pltpu

N_DEV = 32
MASKS = [16, 4, 2, 1, 8]


def _xorval(s: int) -> int:
    v = 0
    for r in range(5):
        if (s >> r) & 1:
            v ^= MASKS[r]
    return v


def kernel(x, w_mat, scale_x, scale_w):
    m_per, k = x.shape
    n_per = w_mat.shape[1]

    x16 = x.astype(jnp.bfloat16)
    x_e = x16[:, 0::2]
    x_o = x16[:, 1::2]
    w8r = w_mat.astype(jnp.float8_e4m3fn).reshape(k // 2, 2 * n_per)

    def body(xe_ref, xo_ref, w_ref, sx_ref, sw_ref, out_ref,
             gath, res_buf, rd_send, rd_recv, res_send, res_recv):
        my = lax.axis_index("i")

        barrier_sem = pltpu.get_barrier_semaphore()
        for m in MASKS:
            pl.semaphore_signal(barrier_sem, inc=1, device_id=(my ^ m,),
                                device_id_type=pl.DeviceIdType.MESH)
        pl.semaphore_wait(barrier_sem, len(MASKS))

        scale = sx_ref[0] * sw_ref[0]
        xe = xe_ref[...]
        xo = xo_ref[...]
        my_rows = pl.ds(my * m_per, m_per)

        def mm(chunk):
            c16 = chunk.astype(jnp.bfloat16)
            acc = jnp.dot(xe, c16[:, :n_per], preferred_element_type=jnp.float32)
            acc += jnp.dot(xo, c16[:, n_per:], preferred_element_type=jnp.float32)
            return jnp.maximum(acc * scale, 0.0)

        gath[0] = w_ref[...]
        out_ref[my_rows, :] = mm(w_ref[...])

        desc_res = [None] * N_DEV

        def do_block(s):
            lbl = _xorval(s)
            o = my ^ lbl
            res_buf[s] = mm(gath[s])
            desc_res[lbl] = pltpu.make_async_remote_copy(
                src_ref=res_buf.at[s],
                dst_ref=out_ref.at[my_rows, :],
                send_sem=res_send.at[lbl],
                recv_sem=res_recv.at[lbl],
                device_id=(o,),
                device_id_type=pl.DeviceIdType.MESH,
            )
            desc_res[lbl].start()

        desc_rd = []
        for r in range(5):
            sz = 1 << r
            d = pltpu.make_async_remote_copy(
                src_ref=gath.at[pl.ds(0, sz)],
                dst_ref=gath.at[pl.ds(sz, sz)],
                send_sem=rd_send.at[r],
                recv_sem=rd_recv.at[r],
                device_id=(my ^ MASKS[r],),
                device_id_type=pl.DeviceIdType.MESH,
            )
            d.start()
            desc_rd.append(d)
            for s in range(sz >> 1 if r else 1, sz):
                do_block(s)
            d.wait_recv()
        for s in range(16, 32):
            do_block(s)

        for lbl in range(1, N_DEV):
            desc_res[lbl].wait_recv()
        for d in desc_rd:
            d.wait_send()
        for lbl in range(1, N_DEV):
            desc_res[lbl].wait_send()

    return pl.pallas_call(
        body,
        out_shape=jax.ShapeDtypeStruct((N_DEV * m_per, n_per), jnp.float32),
        in_specs=[
            pl.BlockSpec(memory_space=pltpu.VMEM),
            pl.BlockSpec(memory_space=pltpu.VMEM),
            pl.BlockSpec(memory_space=pltpu.VMEM),
            pl.BlockSpec(memory_space=pltpu.SMEM),
            pl.BlockSpec(memory_space=pltpu.SMEM),
        ],
        out_specs=pl.BlockSpec(memory_space=pltpu.VMEM),
        scratch_shapes=[
            pltpu.VMEM((N_DEV, k // 2, 2 * n_per), jnp.float8_e4m3fn),
            pltpu.VMEM((N_DEV, m_per, n_per), jnp.float32),
            pltpu.SemaphoreType.DMA((5,)),
            pltpu.SemaphoreType.DMA((5,)),
            pltpu.SemaphoreType.DMA((N_DEV,)),
            pltpu.SemaphoreType.DMA((N_DEV,)),
        ],
        compiler_params=pltpu.CompilerParams(collective_id=0),
    )(x_e, x_o, w8r, scale_x, scale_w)


# device time: 135695 ns/iter; 1.7241x vs baseline; 1.2705x over previous
import jax
import jax.numpy as jnp
from jax import lax
from jax.experimental import pallas as pl
from jax.experimental.pallas import tpu as pltpu

N_DEV = 32
MASKS = [16, 4, 2, 1, 8]


def _xorval(s: int) -> int:
    v = 0
    for r in range(5):
        if (s >> r) & 1:
            v ^= MASKS[r]
    return v


def kernel(x, w_mat, scale_x, scale_w):
    m_per, k = x.shape
    n_per = w_mat.shape[1]
    kh = k // 2

    x16 = x.astype(jnp.bfloat16)
    w8 = w_mat.astype(jnp.float8_e4m3fn)

    def body(x_ref, w_ref, sx_ref, sw_ref, out_ref,
             gath, res_buf, rd_send, rd_recv, res_send, res_recv):
        my = lax.axis_index("i")

        barrier_sem = pltpu.get_barrier_semaphore()
        for m in MASKS:
            pl.semaphore_signal(barrier_sem, inc=1, device_id=(my ^ m,),
                                device_id_type=pl.DeviceIdType.MESH)
        pl.semaphore_wait(barrier_sem, len(MASKS))

        scale = sx_ref[0] * sw_ref[0]
        xv = x_ref[...]
        xe = xv[:, :kh]
        xo = xv[:, kh:]
        my_rows = pl.ds(my * m_per, m_per)

        def mm(chunk):
            c16 = chunk.astype(jnp.bfloat16)
            acc = jnp.dot(xe, c16[:, :n_per], preferred_element_type=jnp.float32)
            acc += jnp.dot(xo, c16[:, n_per:], preferred_element_type=jnp.float32)
            return jnp.maximum(acc * scale, 0.0)

        gath[0, :, :n_per] = w_ref[:kh, :]
        gath[0, :, n_per:] = w_ref[kh:, :]
        out_ref[my_rows, :] = mm(gath[0])

        desc_res = [None] * N_DEV

        def do_block(s):
            lbl = _xorval(s)
            o = my ^ lbl
            res_buf[s] = mm(gath[s])
            desc_res[lbl] = pltpu.make_async_remote_copy(
                src_ref=res_buf.at[s],
                dst_ref=out_ref.at[my_rows, :],
                send_sem=res_send.at[lbl],
                recv_sem=res_recv.at[lbl],
                device_id=(o,),
                device_id_type=pl.DeviceIdType.MESH,
            )
            desc_res[lbl].start()

        desc_rd = []
        for r in range(5):
            sz = 1 << r
            d = pltpu.make_async_remote_copy(
                src_ref=gath.at[pl.ds(0, sz)],
                dst_ref=gath.at[pl.ds(sz, sz)],
                send_sem=rd_send.at[r],
                recv_sem=rd_recv.at[r],
                device_id=(my ^ MASKS[r],),
                device_id_type=pl.DeviceIdType.MESH,
            )
            d.start()
            desc_rd.append(d)
            for s in range(sz >> 1 if r else 1, sz):
                do_block(s)
            d.wait_recv()
        for s in range(16, 32):
            do_block(s)

        for lbl in range(1, N_DEV):
            desc_res[lbl].wait_recv()
        for d in desc_rd:
            d.wait_send()
        for lbl in range(1, N_DEV):
            desc_res[lbl].wait_send()

    return pl.pallas_call(
        body,
        out_shape=jax.ShapeDtypeStruct((N_DEV * m_per, n_per), jnp.float32),
        in_specs=[
            pl.BlockSpec(memory_space=pltpu.VMEM),
            pl.BlockSpec(memory_space=pltpu.VMEM),
            pl.BlockSpec(memory_space=pltpu.SMEM),
            pl.BlockSpec(memory_space=pltpu.SMEM),
        ],
        out_specs=pl.BlockSpec(memory_space=pltpu.VMEM),
        scratch_shapes=[
            pltpu.VMEM((N_DEV, k // 2, 2 * n_per), jnp.float8_e4m3fn),
            pltpu.VMEM((N_DEV, m_per, n_per), jnp.float32),
            pltpu.SemaphoreType.DMA((5,)),
            pltpu.SemaphoreType.DMA((5,)),
            pltpu.SemaphoreType.DMA((N_DEV,)),
            pltpu.SemaphoreType.DMA((N_DEV,)),
        ],
        compiler_params=pltpu.CompilerParams(collective_id=0),
    )(x16, w8, scale_x, scale_w)


# device time: 135396 ns/iter; 1.7279x vs baseline; 1.0022x over previous
import jax
import jax.numpy as jnp
from jax import lax
from jax.experimental import pallas as pl
from jax.experimental.pallas import tpu as pltpu

N_DEV = 32
MASKS = [16, 4, 2, 1, 8]


def _xorval(s: int) -> int:
    v = 0
    for r in range(5):
        if (s >> r) & 1:
            v ^= MASKS[r]
    return v


def kernel(x, w_mat, scale_x, scale_w):
    m_per, k = x.shape
    n_per = w_mat.shape[1]
    kh = k // 2

    x16 = x.astype(jnp.bfloat16)
    w8 = w_mat.astype(jnp.float8_e4m3fn)

    def body(x_ref, w_ref, sx_ref, sw_ref, out_ref,
             gath, res_buf, rd_send, rd_recv, res_send, res_recv):
        my = lax.axis_index("i")

        barrier_sem = pltpu.get_barrier_semaphore()
        for m in MASKS:
            pl.semaphore_signal(barrier_sem, inc=1, device_id=(my ^ m,),
                                device_id_type=pl.DeviceIdType.MESH)
        pl.semaphore_wait(barrier_sem, len(MASKS))

        scale = sx_ref[0] * sw_ref[0]
        xv = x_ref[...]
        xe = xv[:, :kh]
        xo = xv[:, kh:]
        my_rows = pl.ds(my * m_per, m_per)

        def mm(chunk):
            c16 = chunk.astype(jnp.bfloat16)
            acc = jnp.dot(xe, c16[:, :n_per], preferred_element_type=jnp.float32)
            acc += jnp.dot(xo, c16[:, n_per:], preferred_element_type=jnp.float32)
            return jnp.maximum(acc * scale, 0.0)

        gath[0, :, :n_per] = w_ref[:kh, :]
        gath[0, :, n_per:] = w_ref[kh:, :]
        out_ref[my_rows, :] = mm(gath[0])

        desc_res = [None] * N_DEV

        def do_block(s):
            lbl = _xorval(s)
            o = my ^ lbl
            res_buf[s] = mm(gath[s])
            desc_res[lbl] = pltpu.make_async_remote_copy(
                src_ref=res_buf.at[s],
                dst_ref=out_ref.at[my_rows, :],
                send_sem=res_send.at[lbl],
                recv_sem=res_recv.at[lbl],
                device_id=(o,),
                device_id_type=pl.DeviceIdType.MESH,
            )
            desc_res[lbl].start()

        desc_rd = []
        for r in range(4):
            sz = 1 << r
            d = pltpu.make_async_remote_copy(
                src_ref=gath.at[pl.ds(0, sz)],
                dst_ref=gath.at[pl.ds(sz, sz)],
                send_sem=rd_send.at[r],
                recv_sem=rd_recv.at[r],
                device_id=(my ^ MASKS[r],),
                device_id_type=pl.DeviceIdType.MESH,
            )
            d.start()
            desc_rd.append(d)
            for s in range(sz >> 1 if r else 1, sz):
                do_block(s)
            d.wait_recv()
        partner5 = (my ^ MASKS[4],)
        d5a = pltpu.make_async_remote_copy(
            src_ref=gath.at[pl.ds(0, 8)], dst_ref=gath.at[pl.ds(16, 8)],
            send_sem=rd_send.at[4], recv_sem=rd_recv.at[4],
            device_id=partner5, device_id_type=pl.DeviceIdType.MESH,
        )
        d5b = pltpu.make_async_remote_copy(
            src_ref=gath.at[pl.ds(8, 8)], dst_ref=gath.at[pl.ds(24, 8)],
            send_sem=rd_send.at[5], recv_sem=rd_recv.at[5],
            device_id=partner5, device_id_type=pl.DeviceIdType.MESH,
        )
        d5a.start()
        d5b.start()
        desc_rd += [d5a, d5b]
        for s in range(8, 16):
            do_block(s)
        d5a.wait_recv()
        for s in range(16, 24):
            do_block(s)
        d5b.wait_recv()
        for s in range(24, 32):
            do_block(s)

        for lbl in range(1, N_DEV):
            desc_res[lbl].wait_recv()
        for d in desc_rd:
            d.wait_send()
        for lbl in range(1, N_DEV):
            desc_res[lbl].wait_send()

    return pl.pallas_call(
        body,
        out_shape=jax.ShapeDtypeStruct((N_DEV * m_per, n_per), jnp.float32),
        in_specs=[
            pl.BlockSpec(memory_space=pltpu.VMEM),
            pl.BlockSpec(memory_space=pltpu.VMEM),
            pl.BlockSpec(memory_space=pltpu.SMEM),
            pl.BlockSpec(memory_space=pltpu.SMEM),
        ],
        out_specs=pl.BlockSpec(memory_space=pltpu.VMEM),
        scratch_shapes=[
            pltpu.VMEM((N_DEV, k // 2, 2 * n_per), jnp.float8_e4m3fn),
            pltpu.VMEM((N_DEV, m_per, n_per), jnp.float32),
            pltpu.SemaphoreType.DMA((6,)),
            pltpu.SemaphoreType.DMA((6,)),
            pltpu.SemaphoreType.DMA((N_DEV,)),
            pltpu.SemaphoreType.DMA((N_DEV,)),
        ],
        compiler_params=pltpu.CompilerParams(collective_id=0),
    )(x16, w8, scale_x, scale_w)


# device time: 101088 ns/iter; 2.3143x vs baseline; 1.3394x over previous
import jax
import jax.numpy as jnp
from jax import lax
from jax.experimental import pallas as pl
from jax.experimental.pallas import tpu as pltpu

N_DEV = 32
MASKS = [16, 4, 2, 1, 8]


def _xorval(s: int) -> int:
    v = 0
    for r in range(5):
        if (s >> r) & 1:
            v ^= MASKS[r]
    return v


def kernel(x, w_mat, scale_x, scale_w):
    m_per, k = x.shape
    n_per = w_mat.shape[1]
    kh = k // 2

    x16 = x.astype(jnp.bfloat16)
    w8 = w_mat.astype(jnp.float8_e4m3fn)

    def body(x_ref, w_ref, sx_ref, sw_ref, out_ref,
             gath, res_buf, rd_send, rd_recv, res_send, res_recv):
        my = lax.axis_index("i")

        barrier_sem = pltpu.get_barrier_semaphore()
        for m in MASKS:
            pl.semaphore_signal(barrier_sem, inc=1, device_id=(my ^ m,),
                                device_id_type=pl.DeviceIdType.MESH)
        pl.semaphore_wait(barrier_sem, len(MASKS))

        scale = sx_ref[0] * sw_ref[0]
        xv = x_ref[...]
        xe = xv[:, :kh]
        xo = xv[:, kh:]
        my_rows = pl.ds(my * m_per, m_per)

        def mm(chunk):
            c16 = chunk.astype(jnp.bfloat16)
            acc = jnp.dot(xe, c16[:, :n_per], preferred_element_type=jnp.float32)
            acc += jnp.dot(xo, c16[:, n_per:], preferred_element_type=jnp.float32)
            return jnp.maximum(acc * scale, 0.0)

        gath[0, :, :n_per] = w_ref[:kh, :]
        gath[0, :, n_per:] = w_ref[kh:, :]
        out_ref[my_rows, :] = mm(gath[0])

        desc_res = [None] * N_DEV

        def do_block(s):
            lbl = _xorval(s)
            o = my ^ lbl
            res_buf[s] = mm(gath[s])
            desc_res[lbl] = pltpu.make_async_remote_copy(
                src_ref=res_buf.at[s],
                dst_ref=out_ref.at[my_rows, :],
                send_sem=res_send.at[lbl],
                recv_sem=res_recv.at[lbl],
                device_id=(o,),
                device_id_type=pl.DeviceIdType.MESH,
            )
            desc_res[lbl].start()

        def rd_desc(idx, src_lo, src_sz, dst_lo, r):
            return pltpu.make_async_remote_copy(
                src_ref=gath.at[pl.ds(src_lo, src_sz)],
                dst_ref=gath.at[pl.ds(dst_lo, src_sz)],
                send_sem=rd_send.at[idx],
                recv_sem=rd_recv.at[idx],
                device_id=(my ^ MASKS[r],),
                device_id_type=pl.DeviceIdType.MESH,
            )

        f0 = rd_desc(0, 0, 1, 1, 0)
        f1a = rd_desc(1, 0, 1, 2, 1)
        f1b = rd_desc(2, 1, 1, 3, 1)
        f2a = rd_desc(3, 0, 2, 4, 2)
        f2b = rd_desc(4, 2, 2, 6, 2)
        f3a = rd_desc(5, 0, 4, 8, 3)
        f3b = rd_desc(6, 4, 4, 12, 3)
        f4a = rd_desc(7, 0, 8, 16, 4)
        f4b = rd_desc(8, 8, 8, 24, 4)
        desc_rd = [f0, f1a, f1b, f2a, f2b, f3a, f3b, f4a, f4b]

        f0.start()
        f1a.start()
        f0.wait_recv()
        f1b.start()
        f2a.start()
        do_block(1)
        f1a.wait_recv()
        f1b.wait_recv()
        f2b.start()
        f3a.start()
        do_block(2)
        do_block(3)
        f2a.wait_recv()
        f2b.wait_recv()
        f3b.start()
        f4a.start()
        for s in range(4, 8):
            do_block(s)
        f3a.wait_recv()
        f3b.wait_recv()
        f4b.start()
        for s in range(8, 16):
            do_block(s)
        f4a.wait_recv()
        for s in range(16, 24):
            do_block(s)
        f4b.wait_recv()
        for s in range(24, 32):
            do_block(s)

        for lbl in range(1, N_DEV):
            desc_res[lbl].wait_recv()
        for d in desc_rd:
            d.wait_send()
        for lbl in range(1, N_DEV):
            desc_res[lbl].wait_send()

    return pl.pallas_call(
        body,
        out_shape=jax.ShapeDtypeStruct((N_DEV * m_per, n_per), jnp.float32),
        in_specs=[
            pl.BlockSpec(memory_space=pltpu.VMEM),
            pl.BlockSpec(memory_space=pltpu.VMEM),
            pl.BlockSpec(memory_space=pltpu.SMEM),
            pl.BlockSpec(memory_space=pltpu.SMEM),
        ],
        out_specs=pl.BlockSpec(memory_space=pltpu.VMEM),
        scratch_shapes=[
            pltpu.VMEM((N_DEV, k // 2, 2 * n_per), jnp.float8_e4m3fn),
            pltpu.VMEM((N_DEV, m_per, n_per), jnp.float32),
            pltpu.SemaphoreType.DMA((9,)),
            pltpu.SemaphoreType.DMA((9,)),
            pltpu.SemaphoreType.DMA((N_DEV,)),
            pltpu.SemaphoreType.DMA((N_DEV,)),
        ],
        compiler_params=pltpu.CompilerParams(collective_id=0),
    )(x16, w8, scale_x, scale_w)


# device time: 100454 ns/iter; 2.3289x vs baseline; 1.0063x over previous
import jax
import jax.numpy as jnp
from jax import lax
from jax.experimental import pallas as pl
from jax.experimental.pallas import tpu as pltpu

N_DEV = 32
MASKS = [16, 4, 2, 1, 8]


def _xorval(s: int) -> int:
    v = 0
    for r in range(5):
        if (s >> r) & 1:
            v ^= MASKS[r]
    return v


def kernel(x, w_mat, scale_x, scale_w):
    m_per, k = x.shape
    n_per = w_mat.shape[1]
    kh = k // 2

    x16 = x.astype(jnp.bfloat16)
    w8 = w_mat.astype(jnp.float8_e4m3fn)

    def body(x_ref, w_ref, sx_ref, sw_ref, out_ref,
             gath, res_buf, rd_send, rd_recv, res_send, res_recv):
        my = lax.axis_index("i")

        barrier_sem = pltpu.get_barrier_semaphore()
        for m in MASKS:
            pl.semaphore_signal(barrier_sem, inc=1, device_id=(my ^ m,),
                                device_id_type=pl.DeviceIdType.MESH)
        pl.semaphore_wait(barrier_sem, len(MASKS))

        scale = sx_ref[0] * sw_ref[0]
        xv = x_ref[...]
        xe = xv[:, :kh]
        xo = xv[:, kh:]
        my_rows = pl.ds(my * m_per, m_per)

        def mm(chunk):
            c16 = chunk.astype(jnp.bfloat16)
            acc = jnp.dot(xe, c16[:, :n_per], preferred_element_type=jnp.float32)
            acc += jnp.dot(xo, c16[:, n_per:], preferred_element_type=jnp.float32)
            return jnp.maximum(acc * scale, 0.0)

        gath[0, :, :n_per] = w_ref[:kh, :]
        gath[0, :, n_per:] = w_ref[kh:, :]
        out_ref[my_rows, :] = mm(gath[0])

        desc_res = [None] * N_DEV

        def do_block(s):
            lbl = _xorval(s)
            o = my ^ lbl
            res_buf[s] = mm(gath[s])
            desc_res[lbl] = pltpu.make_async_remote_copy(
                src_ref=res_buf.at[s],
                dst_ref=out_ref.at[my_rows, :],
                send_sem=res_send.at[lbl],
                recv_sem=res_recv.at[lbl],
                device_id=(o,),
                device_id_type=pl.DeviceIdType.MESH,
            )
            desc_res[lbl].start()

        def rd_desc(idx, src_lo, src_sz, dst_lo, r):
            return pltpu.make_async_remote_copy(
                src_ref=gath.at[pl.ds(src_lo, src_sz)],
                dst_ref=gath.at[pl.ds(dst_lo, src_sz)],
                send_sem=rd_send.at[idx],
                recv_sem=rd_recv.at[idx],
                device_id=(my ^ MASKS[r],),
                device_id_type=pl.DeviceIdType.MESH,
            )

        f0 = rd_desc(0, 0, 1, 1, 0)
        f1a = rd_desc(1, 0, 1, 2, 1)
        f1b = rd_desc(2, 1, 1, 3, 1)
        f2a = rd_desc(3, 0, 2, 4, 2)
        f2b = rd_desc(4, 2, 2, 6, 2)
        f3a = rd_desc(5, 0, 4, 8, 3)
        f3b = rd_desc(6, 4, 4, 12, 3)
        f4q = [rd_desc(7 + q, 4 * q, 4, 16 + 4 * q, 4) for q in range(4)]
        desc_rd = [f0, f1a, f1b, f2a, f2b, f3a, f3b] + f4q

        f0.start()
        f1a.start()
        f0.wait_recv()
        f1b.start()
        f2a.start()
        do_block(1)
        f1a.wait_recv()
        f1b.wait_recv()
        f2b.start()
        f3a.start()
        do_block(2)
        do_block(3)
        f2a.wait_recv()
        f2b.wait_recv()
        f3b.start()
        f4q[0].start()
        f4q[1].start()
        for s in range(4, 8):
            do_block(s)
        f3a.wait_recv()
        f3b.wait_recv()
        f4q[2].start()
        f4q[3].start()
        for s in range(8, 16):
            do_block(s)
        for q in range(4):
            f4q[q].wait_recv()
            for s in range(16 + 4 * q, 20 + 4 * q):
                do_block(s)

        for lbl in range(1, N_DEV):
            desc_res[lbl].wait_recv()
        for d in desc_rd:
            d.wait_send()
        for lbl in range(1, N_DEV):
            desc_res[lbl].wait_send()

    return pl.pallas_call(
        body,
        out_shape=jax.ShapeDtypeStruct((N_DEV * m_per, n_per), jnp.float32),
        in_specs=[
            pl.BlockSpec(memory_space=pltpu.VMEM),
            pl.BlockSpec(memory_space=pltpu.VMEM),
            pl.BlockSpec(memory_space=pltpu.SMEM),
            pl.BlockSpec(memory_space=pltpu.SMEM),
        ],
        out_specs=pl.BlockSpec(memory_space=pltpu.VMEM),
        scratch_shapes=[
            pltpu.VMEM((N_DEV, k // 2, 2 * n_per), jnp.float8_e4m3fn),
            pltpu.VMEM((N_DEV, m_per, n_per), jnp.float32),
            pltpu.SemaphoreType.DMA((11,)),
            pltpu.SemaphoreType.DMA((11,)),
            pltpu.SemaphoreType.DMA((N_DEV,)),
            pltpu.SemaphoreType.DMA((N_DEV,)),
        ],
        compiler_params=pltpu.CompilerParams(collective_id=0),
    )(x16, w8, scale_x, scale_w)


# device time: 90735 ns/iter; 2.5784x vs baseline; 1.1071x over previous
import jax
import jax.numpy as jnp
from jax import lax
from jax.experimental import pallas as pl
from jax.experimental.pallas import tpu as pltpu

N_DEV = 32
MASKS = [16, 4, 2, 1, 8]


def _xorval(s: int) -> int:
    v = 0
    for r in range(5):
        if (s >> r) & 1:
            v ^= MASKS[r]
    return v


def kernel(x, w_mat, scale_x, scale_w):
    m_per, k = x.shape
    n_per = w_mat.shape[1]
    kh = k // 2

    x16 = x.astype(jnp.bfloat16)
    w8 = w_mat.astype(jnp.float8_e4m3fn)

    def body(x_ref, w_ref, sx_ref, sw_ref, out_ref,
             gath, res_pack, res_stage, rd_send, rd_recv, res_send, res_recv):
        my = lax.axis_index("i")

        barrier_sem = pltpu.get_barrier_semaphore()
        for m in MASKS:
            pl.semaphore_signal(barrier_sem, inc=1, device_id=(my ^ m,),
                                device_id_type=pl.DeviceIdType.MESH)
        pl.semaphore_wait(barrier_sem, len(MASKS))

        scale = sx_ref[0] * sw_ref[0]
        xv = x_ref[...]
        xe = xv[:, :kh]
        xo = xv[:, kh:]
        my_rows = pl.ds(my * m_per, m_per)

        def mm(chunk):
            c16 = chunk.astype(jnp.bfloat16)
            acc = jnp.dot(xe, c16[:, :n_per], preferred_element_type=jnp.float32)
            acc += jnp.dot(xo, c16[:, n_per:], preferred_element_type=jnp.float32)
            return jnp.maximum(acc * scale, 0.0)

        gath[0, :, :n_per] = w_ref[:kh, :]
        gath[0, :, n_per:] = w_ref[kh:, :]
        out_ref[my_rows, :] = mm(gath[0])

        desc_res = [None] * N_DEV

        mh = m_per // 2

        def do_block(s):
            lbl = _xorval(s)
            o = my ^ lbl
            r16 = mm(gath[s]).astype(jnp.bfloat16)
            res_pack[s, :, :n_per] = r16[:mh, :]
            res_pack[s, :, n_per:] = r16[mh:, :]
            desc_res[lbl] = pltpu.make_async_remote_copy(
                src_ref=res_pack.at[s],
                dst_ref=res_stage.at[lbl],
                send_sem=res_send.at[lbl],
                recv_sem=res_recv.at[lbl],
                device_id=(o,),
                device_id_type=pl.DeviceIdType.MESH,
            )
            desc_res[lbl].start()

        def rd_desc(idx, src_lo, src_sz, dst_lo, r):
            return pltpu.make_async_remote_copy(
                src_ref=gath.at[pl.ds(src_lo, src_sz)],
                dst_ref=gath.at[pl.ds(dst_lo, src_sz)],
                send_sem=rd_send.at[idx],
                recv_sem=rd_recv.at[idx],
                device_id=(my ^ MASKS[r],),
                device_id_type=pl.DeviceIdType.MESH,
            )

        f0 = rd_desc(0, 0, 1, 1, 0)
        f1a = rd_desc(1, 0, 1, 2, 1)
        f1b = rd_desc(2, 1, 1, 3, 1)
        f2a = rd_desc(3, 0, 2, 4, 2)
        f2b = rd_desc(4, 2, 2, 6, 2)
        f3a = rd_desc(5, 0, 4, 8, 3)
        f3b = rd_desc(6, 4, 4, 12, 3)
        f4q = [rd_desc(7 + q, 4 * q, 4, 16 + 4 * q, 4) for q in range(4)]
        desc_rd = [f0, f1a, f1b, f2a, f2b, f3a, f3b] + f4q

        f0.start()
        f1a.start()
        f0.wait_recv()
        f1b.start()
        f2a.start()
        do_block(1)
        f1a.wait_recv()
        f1b.wait_recv()
        f2b.start()
        f3a.start()
        do_block(2)
        do_block(3)
        f2a.wait_recv()
        f2b.wait_recv()
        f3b.start()
        f4q[0].start()
        f4q[1].start()
        for s in range(4, 8):
            do_block(s)
        f3a.wait_recv()
        f3b.wait_recv()
        f4q[2].start()
        f4q[3].start()
        for s in range(8, 16):
            do_block(s)
        for q in range(4):
            f4q[q].wait_recv()
            for s in range(16 + 4 * q, 20 + 4 * q):
                do_block(s)

        for lbl in range(1, N_DEV):
            desc_res[lbl].wait_recv()
            base = (my ^ lbl) * m_per
            st = res_stage[lbl]
            out_ref[pl.ds(base, mh), :] = st[:, :n_per].astype(jnp.float32)
            out_ref[pl.ds(base + mh, mh), :] = st[:, n_per:].astype(jnp.float32)
        for d in desc_rd:
            d.wait_send()
        for lbl in range(1, N_DEV):
            desc_res[lbl].wait_send()

    return pl.pallas_call(
        body,
        out_shape=jax.ShapeDtypeStruct((N_DEV * m_per, n_per), jnp.float32),
        in_specs=[
            pl.BlockSpec(memory_space=pltpu.VMEM),
            pl.BlockSpec(memory_space=pltpu.VMEM),
            pl.BlockSpec(memory_space=pltpu.SMEM),
            pl.BlockSpec(memory_space=pltpu.SMEM),
        ],
        out_specs=pl.BlockSpec(memory_space=pltpu.VMEM),
        scratch_shapes=[
            pltpu.VMEM((N_DEV, k // 2, 2 * n_per), jnp.float8_e4m3fn),
            pltpu.VMEM((N_DEV, m_per // 2, 2 * n_per), jnp.bfloat16),
            pltpu.VMEM((N_DEV, m_per // 2, 2 * n_per), jnp.bfloat16),
            pltpu.SemaphoreType.DMA((11,)),
            pltpu.SemaphoreType.DMA((11,)),
            pltpu.SemaphoreType.DMA((N_DEV,)),
            pltpu.SemaphoreType.DMA((N_DEV,)),
        ],
        compiler_params=pltpu.CompilerParams(collective_id=0),
    )(x16, w8, scale_x, scale_w)
